# speedup vs baseline: 4.5404x; 4.5404x over previous
import functools
import jax
import jax.numpy as jnp
from jax import lax
from jax.experimental import pallas as pl
from jax.experimental.pallas import tpu as pltpu
from jax.experimental.pallas import tpu_sc as plsc

B, L, H = 4096, 50, 128
NB = 4


def kernel(x, table1, table2):
    info = plsc.get_sparse_core_info()
    nw = info.num_cores * info.num_subcores
    bpw = B // nw
    mesh = plsc.VectorSubcoreMesh(core_axis_name="c", subcore_axis_name="s")

    @functools.partial(
        pl.kernel,
        mesh=mesh,
        out_type=jax.ShapeDtypeStruct((B, 2, L, H), jnp.float32),
        scratch_types=[
            pltpu.VMEM((NB, L), jnp.int32),
            pltpu.VMEM((NB, 2, L, H), jnp.float32),
            pltpu.SemaphoreType.DMA,
        ],
    )
    def run(x_hbm, t1_hbm, t2_hbm, out_hbm, idx_v, buf_v, sem):
        wid = lax.axis_index("s") * info.num_cores + lax.axis_index("c")

        def body(g, carry):
            b0 = wid * bpw + g * NB
            pltpu.sync_copy(x_hbm.at[pl.ds(b0, NB)], idx_v)
            for nb in range(NB):
                pltpu.async_copy(t1_hbm.at[idx_v.at[nb]], buf_v.at[nb, 0], sem)
                pltpu.async_copy(t2_hbm.at[idx_v.at[nb]], buf_v.at[nb, 1], sem)
            for nb in range(NB):
                pltpu.make_async_copy(t1_hbm.at[idx_v.at[nb]], buf_v.at[nb, 0], sem).wait()
                pltpu.make_async_copy(t2_hbm.at[idx_v.at[nb]], buf_v.at[nb, 1], sem).wait()
            pltpu.sync_copy(buf_v, out_hbm.at[pl.ds(b0, NB)])
            return carry

        lax.fori_loop(0, bpw // NB, body, 0)

    return run(x, table1, table2)


# speedup vs baseline: 4.7653x; 1.0495x over previous
import functools
import jax
import jax.numpy as jnp
from jax import lax
from jax.experimental import pallas as pl
from jax.experimental.pallas import tpu as pltpu
from jax.experimental.pallas import tpu_sc as plsc

B, L, H = 4096, 50, 128
NB = 4


def kernel(x, table1, table2):
    info = plsc.get_sparse_core_info()
    nw = info.num_cores * info.num_subcores
    bpw = B // nw
    nchunks = bpw // NB
    npairs = nchunks // 2
    mesh = plsc.VectorSubcoreMesh(core_axis_name="c", subcore_axis_name="s")

    @functools.partial(
        pl.kernel,
        mesh=mesh,
        out_type=jax.ShapeDtypeStruct((B, 2, L, H), jnp.float32),
        scratch_types=[
            pltpu.VMEM((bpw, L), jnp.int32),
            pltpu.VMEM((2, NB, 2, L, H), jnp.float32),
            pltpu.SemaphoreType.DMA,
            pltpu.SemaphoreType.DMA,
            pltpu.SemaphoreType.DMA,
            pltpu.SemaphoreType.DMA,
        ],
    )
    def run(x_hbm, t1_hbm, t2_hbm, out_hbm, idx_v, buf_v, g0, g1, w0, w1):
        wid = lax.axis_index("s") * info.num_cores + lax.axis_index("c")
        b0w = wid * bpw
        gsem = (g0, g1)
        wsem = (w0, w1)

        pltpu.sync_copy(x_hbm.at[pl.ds(b0w, bpw)], idx_v)

        def issue_gathers(g, slot):
            for nb in range(NB):
                row = g * NB + nb
                pltpu.async_copy(t1_hbm.at[idx_v.at[row]], buf_v.at[slot, nb, 0], gsem[slot])
                pltpu.async_copy(t2_hbm.at[idx_v.at[row]], buf_v.at[slot, nb, 1], gsem[slot])

        def drain_gathers(slot):
            for nb in range(NB):
                pltpu.make_async_copy(t1_hbm.at[idx_v.at[0]], buf_v.at[slot, nb, 0], gsem[slot]).wait()
                pltpu.make_async_copy(t2_hbm.at[idx_v.at[0]], buf_v.at[slot, nb, 1], gsem[slot]).wait()

        def issue_write(g, slot):
            b0 = b0w + g * NB
            pltpu.async_copy(buf_v.at[slot], out_hbm.at[pl.ds(b0, NB)], wsem[slot])

        def wait_write(slot):
            pltpu.make_async_copy(buf_v.at[slot], out_hbm.at[pl.ds(b0w, NB)], wsem[slot]).wait()

        issue_gathers(0, 0)
        issue_gathers(1, 1)

        def body(i, carry):
            drain_gathers(0)
            issue_write(2 * i, 0)
            drain_gathers(1)
            issue_write(2 * i + 1, 1)

            @pl.when(i < npairs - 1)
            def _():
                wait_write(0)
                issue_gathers(2 * i + 2, 0)
                wait_write(1)
                issue_gathers(2 * i + 3, 1)

            return carry

        lax.fori_loop(0, npairs, body, 0)
        wait_write(0)
        wait_write(1)

    return run(x, table1, table2)


# speedup vs baseline: 4.7663x; 1.0002x over previous
import functools
import jax
import jax.numpy as jnp
from jax import lax
from jax.experimental import pallas as pl
from jax.experimental.pallas import tpu as pltpu
from jax.experimental.pallas import tpu_sc as plsc

B, L, H = 4096, 50, 128
NB = 4


def kernel(x, table1, table2):
    info = plsc.get_sparse_core_info()
    nw = info.num_cores * info.num_subcores
    bpw = B // nw
    nchunks = bpw // NB
    npairs = nchunks // 2
    mesh = plsc.VectorSubcoreMesh(core_axis_name="c", subcore_axis_name="s")

    @functools.partial(
        pl.kernel,
        mesh=mesh,
        out_type=jax.ShapeDtypeStruct((B, 2, L, H), jnp.float32),
        scratch_types=[
            pltpu.VMEM((bpw, L), jnp.int32),
            pltpu.VMEM((2, NB, 2, L, H), jnp.float32),
            pltpu.SemaphoreType.DMA,
            pltpu.SemaphoreType.DMA,
            pltpu.SemaphoreType.DMA,
            pltpu.SemaphoreType.DMA,
        ],
        compiler_params=pltpu.CompilerParams(use_tc_tiling_on_sc=True),
    )
    def run(x_hbm, t1_hbm, t2_hbm, out_hbm, idx_v, buf_v, g0, g1, w0, w1):
        wid = lax.axis_index("s") * info.num_cores + lax.axis_index("c")
        b0w = wid * bpw
        gsem = (g0, g1)
        wsem = (w0, w1)

        pltpu.sync_copy(x_hbm.at[pl.ds(b0w, bpw)], idx_v)

        def issue_gathers(g, slot):
            for nb in range(NB):
                row = g * NB + nb
                pltpu.async_copy(t1_hbm.at[idx_v.at[row]], buf_v.at[slot, nb, 0], gsem[slot])
                pltpu.async_copy(t2_hbm.at[idx_v.at[row]], buf_v.at[slot, nb, 1], gsem[slot])

        def drain_gathers(slot):
            for nb in range(NB):
                pltpu.make_async_copy(t1_hbm.at[idx_v.at[0]], buf_v.at[slot, nb, 0], gsem[slot]).wait()
                pltpu.make_async_copy(t2_hbm.at[idx_v.at[0]], buf_v.at[slot, nb, 1], gsem[slot]).wait()

        def issue_write(g, slot):
            b0 = b0w + g * NB
            pltpu.async_copy(buf_v.at[slot], out_hbm.at[pl.ds(b0, NB)], wsem[slot])

        def wait_write(slot):
            pltpu.make_async_copy(buf_v.at[slot], out_hbm.at[pl.ds(b0w, NB)], wsem[slot]).wait()

        issue_gathers(0, 0)
        issue_gathers(1, 1)

        def body(i, carry):
            drain_gathers(0)
            issue_write(2 * i, 0)
            drain_gathers(1)
            issue_write(2 * i + 1, 1)

            @pl.when(i < npairs - 1)
            def _():
                wait_write(0)
                issue_gathers(2 * i + 2, 0)
                wait_write(1)
                issue_gathers(2 * i + 3, 1)

            return carry

        lax.fori_loop(0, npairs, body, 0)
        wait_write(0)
        wait_write(1)

    return run(x, table1, table2)


# speedup vs baseline: 12.0273x; 2.5234x over previous
import functools
import jax
import jax.numpy as jnp
from jax import lax
from jax.experimental import pallas as pl
from jax.experimental.pallas import tpu as pltpu
from jax.experimental.pallas import tpu_sc as plsc

B, L, H = 4096, 50, 128
N = B * L
CH = 80


def kernel(x, table1, table2):
    info = plsc.get_sparse_core_info()
    nw = info.num_cores * info.num_subcores
    npw = N // nw
    nchunks = npw // CH
    npairs = nchunks // 2
    mesh = plsc.VectorSubcoreMesh(core_axis_name="c", subcore_axis_name="s")

    @functools.partial(
        pl.kernel,
        mesh=mesh,
        out_type=jax.ShapeDtypeStruct((2 * N, H), jnp.float32),
        scratch_types=[
            pltpu.VMEM((npw // CH, CH), jnp.int32),
            pltpu.VMEM((2, 2, CH), jnp.int32),
            pltpu.VMEM((2, 2, CH, H), jnp.float32),
            pltpu.SemaphoreType.DMA,
            pltpu.SemaphoreType.DMA,
            pltpu.SemaphoreType.DMA,
            pltpu.SemaphoreType.DMA,
        ],
    )
    def run(x_hbm, t1_hbm, t2_hbm, out_hbm, xidx_v, didx_v, buf_v, g0, g1, w0, w1):
        wid = lax.axis_index("s") * info.num_cores + lax.axis_index("c")
        r0w = wid * npw
        gsem = (g0, g1)
        wsem = (w0, w1)
        lanes = lax.iota(jnp.int32, 16)

        pltpu.sync_copy(x_hbm.at[pl.ds(wid * (npw // CH), npw // CH)], xidx_v)

        def issue_gathers(c, slot):
            pltpu.async_copy(t1_hbm.at[xidx_v.at[c]], buf_v.at[slot, 0], gsem[slot])
            pltpu.async_copy(t2_hbm.at[xidx_v.at[c]], buf_v.at[slot, 1], gsem[slot])

        def drain_gathers(slot):
            pltpu.make_async_copy(t1_hbm.at[xidx_v.at[0]], buf_v.at[slot, 0], gsem[slot]).wait()
            pltpu.make_async_copy(t2_hbm.at[xidx_v.at[0]], buf_v.at[slot, 1], gsem[slot]).wait()

        def fill_dst_idx(c, slot):
            base = 2 * (r0w + c * CH)
            for t in range(2):
                for k in range(CH // 16):
                    didx_v[slot, t, pl.ds(16 * k, 16)] = base + 2 * (16 * k + lanes) + t

        def issue_scatters(slot):
            pltpu.async_copy(buf_v.at[slot, 0], out_hbm.at[didx_v.at[slot, 0]], wsem[slot])
            pltpu.async_copy(buf_v.at[slot, 1], out_hbm.at[didx_v.at[slot, 1]], wsem[slot])

        def wait_scatters(slot):
            pltpu.make_async_copy(buf_v.at[slot, 0], out_hbm.at[didx_v.at[slot, 0]], wsem[slot]).wait()
            pltpu.make_async_copy(buf_v.at[slot, 1], out_hbm.at[didx_v.at[slot, 1]], wsem[slot]).wait()

        issue_gathers(0, 0)
        issue_gathers(1, 1)

        def body(i, carry):
            c = 2 * i
            drain_gathers(0)
            fill_dst_idx(c, 0)
            issue_scatters(0)
            drain_gathers(1)
            fill_dst_idx(c + 1, 1)
            issue_scatters(1)

            @pl.when(i < npairs - 1)
            def _():
                wait_scatters(0)
                issue_gathers(c + 2, 0)
                wait_scatters(1)
                issue_gathers(c + 3, 1)

            return carry

        lax.fori_loop(0, npairs, body, 0)
        wait_scatters(0)
        wait_scatters(1)

    flat = run(x.reshape(N // CH, CH), table1, table2)
    return flat.reshape(B, L, 2, H).transpose(0, 2, 1, 3)
